# baseline (speedup 1.0000x reference)
import functools

import jax
import jax.numpy as jnp
import numpy as np
from jax import lax
from jax.experimental import pallas as pl
from jax.experimental.pallas import tpu as pltpu
from jax.experimental.pallas import tpu_sc as plsc

_INTERPRET = False

B = 4096
E = 16
CAP = 320
NROWS = E * CAP
BUF_ROWS = 17 * CAP
DUMMY = NROWS
D = 2048
HIGH = jax.lax.Precision.DEFAULT
EXACT = jax.lax.Precision.HIGHEST


def _build_w1p(conv1_w):
    W = jnp.zeros((5, 3, 32, 2, 6, 14), jnp.float32)
    jp = np.arange(14)
    for par in range(2):
        for dj in range(5):
            vals = conv1_w[:, :, :, dj].transpose(2, 1, 0)
            vals = jnp.broadcast_to(vals[None], (14, 5, 3, 6))
            W = W.at[:, :, 2 * jp + par + dj, par, :, jp].set(vals)
    return W.reshape(480, 168)


def _build_w2p(conv2_w):
    W = jnp.zeros((5, 6, 14, 2, 16, 5), jnp.float32)
    jp = np.arange(5)
    for par in range(2):
        for dj in range(5):
            vals = conv2_w[:, :, :, dj].transpose(2, 1, 0)
            vals = jnp.broadcast_to(vals[None], (5, 5, 6, 16))
            W = W.at[:, :, 2 * jp + par + dj, par, :, jp].set(vals)
    return W.reshape(420, 160)


def _conv_body(x0_ref, x1_ref, x2_ref, x3_ref, w1_ref, b1_ref, w2_ref,
               b2_ref, o_ref):
    n = x0_ref.shape[0]
    phases = [x0_ref[...], x1_ref[...], x2_ref[...], x3_ref[...]]
    row_blocks = []
    for par1 in range(2):
        for par2 in range(2):
            sl = []
            for di in range(5):
                t = 2 * par2 + par1 + di
                q, off = t % 4, t // 4
                sl.append(phases[q][:, off:off + 7, :])
            row_blocks.append(jnp.concatenate(sl, axis=2))
    p1 = jnp.concatenate(row_blocks, axis=1)
    y1 = jnp.dot(p1.reshape(n * 28, 480), w1_ref[...], precision=HIGH,
                 preferred_element_type=jnp.float32) + b1_ref[...]
    y1 = jnp.maximum(y1, 0.0)
    y1 = jnp.maximum(y1[:, :84], y1[:, 84:])
    y1 = y1.reshape(n, 28, 84)
    h1 = jnp.maximum(y1[:, :14, :], y1[:, 14:, :])
    pb = []
    for par in range(2):
        sl = []
        for di in range(5):
            t = par + di
            q, off = t % 2, t // 2
            sl.append(h1[:, 7 * q + off:7 * q + off + 5, :])
        pb.append(jnp.concatenate(sl, axis=2))
    p2 = jnp.concatenate(pb, axis=1)
    y2 = jnp.dot(p2.reshape(n * 10, 420), w2_ref[...], precision=HIGH,
                 preferred_element_type=jnp.float32) + b2_ref[...]
    y2 = jnp.maximum(y2, 0.0)
    y2 = jnp.maximum(y2[:, :80], y2[:, 80:])
    y2 = y2.reshape(n, 10, 80)
    h2 = jnp.maximum(y2[:, :5, :], y2[:, 5:, :])
    o_ref[...] = h2.reshape(n, 400)


def _conv_trunk(xp, w1p, b1p, w2p, b2p):
    nb = 128
    return pl.pallas_call(
        _conv_body,
        grid=(B // nb,),
        in_specs=[
            pl.BlockSpec((nb, 8, 96), lambda i: (i, 0, 0)),
            pl.BlockSpec((nb, 8, 96), lambda i: (i, 0, 0)),
            pl.BlockSpec((nb, 8, 96), lambda i: (i, 0, 0)),
            pl.BlockSpec((nb, 8, 96), lambda i: (i, 0, 0)),
            pl.BlockSpec((480, 168), lambda i: (0, 0)),
            pl.BlockSpec((1, 168), lambda i: (0, 0)),
            pl.BlockSpec((420, 160), lambda i: (0, 0)),
            pl.BlockSpec((1, 160), lambda i: (0, 0)),
        ],
        out_specs=pl.BlockSpec((nb, 400), lambda i: (i, 0)),
        out_shape=jax.ShapeDtypeStruct((B, 400), jnp.float32),
        interpret=_INTERPRET,
    )(xp[0], xp[1], xp[2], xp[3], w1p, b1p, w2p, b2p)


def _mm_relu_body(x_ref, w_ref, b_ref, o_ref):
    y = jnp.dot(x_ref[...], w_ref[...], precision=HIGH,
                preferred_element_type=jnp.float32) + b_ref[...]
    o_ref[...] = jnp.maximum(y, 0.0)


def _fc1(feat, w, b):
    nb = 512
    return pl.pallas_call(
        _mm_relu_body,
        grid=(B // nb,),
        in_specs=[
            pl.BlockSpec((nb, 400), lambda i: (i, 0)),
            pl.BlockSpec((400, 4096), lambda i: (0, 0)),
            pl.BlockSpec((1, 4096), lambda i: (0, 0)),
        ],
        out_specs=pl.BlockSpec((nb, 4096), lambda i: (i, 0)),
        out_shape=jax.ShapeDtypeStruct((B, 4096), jnp.float32),
        interpret=_INTERPRET,
    )(feat, w, b)


def _fc2_body(x_ref, w_ref, b_ref, o_ref):
    y = jnp.dot(x_ref[...], w_ref[...], precision=HIGH,
                preferred_element_type=jnp.float32) + b_ref[...]
    o_ref[...] = jnp.maximum(y, 0.0)


def _fc2(h1, w, b):
    nb, nc = 512, 512
    return pl.pallas_call(
        _fc2_body,
        grid=(B // nb, D // nc),
        in_specs=[
            pl.BlockSpec((nb, 4096), lambda i, j: (i, 0)),
            pl.BlockSpec((4096, nc), lambda i, j: (0, j)),
            pl.BlockSpec((1, nc), lambda i, j: (0, j)),
        ],
        out_specs=pl.BlockSpec((nb, nc), lambda i, j: (i, j)),
        out_shape=jax.ShapeDtypeStruct((B, D), jnp.float32),
        interpret=_INTERPRET,
    )(h1, w, b)


def _route_body(h_ref, gw_ref, ss_ref, sg_ref, cf_ref, kp_ref, carry):
    @pl.when(pl.program_id(0) == 0)
    def _init():
        carry[...] = jnp.zeros_like(carry)

    nb = h_ref.shape[0]
    logits = jnp.dot(h_ref[...], gw_ref[...], precision=HIGH,
                     preferred_element_type=jnp.float32)
    m = jnp.max(logits, axis=-1, keepdims=True)
    p = jnp.exp(logits - m)
    gate_val = 1.0 / jnp.sum(p, axis=-1, keepdims=True)
    eio = lax.broadcasted_iota(jnp.int32, (nb, E), 1)
    idx = jnp.min(jnp.where(logits == m, eio, E), axis=-1,
                  keepdims=True)
    onehot = (eio == idx).astype(jnp.float32)
    ri = lax.broadcasted_iota(jnp.int32, (nb, nb), 0)
    ci = lax.broadcasted_iota(jnp.int32, (nb, nb), 1)
    tri = (ri >= ci).astype(jnp.float32)
    cum = jnp.dot(tri, onehot, precision=EXACT,
                  preferred_element_type=jnp.float32) + carry[0:1, 0:E]
    loc = jnp.sum(cum * onehot, axis=-1, keepdims=True) - 1.0
    keep = (loc < CAP).astype(jnp.float32)
    locc = jnp.clip(loc, 0.0, CAP - 1.0).astype(jnp.int32)
    slot_g = idx * CAP + locc
    slot_s = jnp.where(keep > 0.0, slot_g, DUMMY)
    ss_ref[...] = slot_s
    sg_ref[...] = slot_g
    cf_ref[...] = gate_val * keep
    kp_ref[...] = keep
    carry[0:1, 0:E] += jnp.sum(onehot, axis=0, keepdims=True)


def _route(h, gate_w):
    nb = 512
    return pl.pallas_call(
        _route_body,
        grid=(B // nb,),
        in_specs=[
            pl.BlockSpec((nb, D), lambda i: (i, 0)),
            pl.BlockSpec((D, E), lambda i: (0, 0)),
        ],
        out_specs=[
            pl.BlockSpec((nb, 1), lambda i: (i, 0)),
            pl.BlockSpec((nb, 1), lambda i: (i, 0)),
            pl.BlockSpec((nb, 1), lambda i: (i, 0)),
            pl.BlockSpec((nb, 1), lambda i: (i, 0)),
        ],
        out_shape=[
            jax.ShapeDtypeStruct((B, 1), jnp.int32),
            jax.ShapeDtypeStruct((B, 1), jnp.int32),
            jax.ShapeDtypeStruct((B, 1), jnp.float32),
            jax.ShapeDtypeStruct((B, 1), jnp.float32),
        ],
        scratch_shapes=[pltpu.VMEM((8, 128), jnp.float32)],
        interpret=_INTERPRET,
    )(h, gate_w)


def _expert_body(x_ref, w_ref, b_ref, o_ref):
    w = w_ref[...].reshape(D, D)
    y = lax.dot_general(x_ref[...], w, (((1,), (1,)), ((), ())),
                        precision=HIGH, preferred_element_type=jnp.float32)
    o_ref[...] = y + b_ref[...].reshape(1, D)


def _experts(buf, expert_w, expert_b3):
    return pl.pallas_call(
        _expert_body,
        grid=(E,),
        in_specs=[
            pl.BlockSpec((CAP, D), lambda e: (e, 0)),
            pl.BlockSpec((1, D, D), lambda e: (e, 0, 0)),
            pl.BlockSpec((1, 1, D), lambda e: (e, 0, 0)),
        ],
        out_specs=pl.BlockSpec((CAP, D), lambda e: (e, 0)),
        out_shape=jax.ShapeDtypeStruct((NROWS, D), jnp.float32),
        interpret=_INTERPRET,
    )(buf, expert_w, expert_b3)


def _fc4_body(g_ref, cf_ref, kp_ref, w_ref, b_ref, o_ref):
    x = jnp.where(kp_ref[...] > 0.0, g_ref[...] * cf_ref[...], 0.0)
    o_ref[...] = jnp.dot(x, w_ref[...], precision=HIGH,
                         preferred_element_type=jnp.float32) + b_ref[...]


def _fc4(gathered, coef, keep, w, b):
    nb = 512
    return pl.pallas_call(
        _fc4_body,
        grid=(B // nb,),
        in_specs=[
            pl.BlockSpec((nb, D), lambda i: (i, 0)),
            pl.BlockSpec((nb, 1), lambda i: (i, 0)),
            pl.BlockSpec((nb, 1), lambda i: (i, 0)),
            pl.BlockSpec((D, 10), lambda i: (0, 0)),
            pl.BlockSpec((1, 10), lambda i: (0, 0)),
        ],
        out_specs=pl.BlockSpec((nb, 10), lambda i: (i, 0)),
        out_shape=jax.ShapeDtypeStruct((B, 10), jnp.float32),
        interpret=_INTERPRET,
    )(gathered, coef, keep, w, b)


_CHUNK = 32


def _sc_dispatch(h, slot_s):
    if _INTERPRET:
        return jnp.zeros((BUF_ROWS, D), jnp.float32).at[slot_s].set(h)
    info = plsc.get_sparse_core_info()
    nw = info.num_cores * info.num_subcores
    per_w = B // nw
    mesh = plsc.VectorSubcoreMesh(core_axis_name="c", subcore_axis_name="s")

    @functools.partial(
        pl.kernel, mesh=mesh,
        out_type=jax.ShapeDtypeStruct((BUF_ROWS, D), jnp.float32),
        scratch_types=[
            pltpu.VMEM((_CHUNK,), jnp.int32),
            pltpu.VMEM((_CHUNK, D), jnp.float32),
            pltpu.SemaphoreType.DMA,
        ],
    )
    def k(h_hbm, slot_hbm, buf_hbm, idx_v, rows_v, sem):
        wid = lax.axis_index("s") * info.num_cores + lax.axis_index("c")
        for c in range(per_w // _CHUNK):
            base = wid * per_w + c * _CHUNK
            pltpu.sync_copy(slot_hbm.at[pl.ds(base, _CHUNK)], idx_v)
            pltpu.sync_copy(h_hbm.at[pl.ds(base, _CHUNK)], rows_v)
            pltpu.async_copy(rows_v, buf_hbm.at[idx_v], sem).wait()

    return k(h, slot_s)


def _sc_combine(flat, slot_g):
    if _INTERPRET:
        return flat[slot_g]
    info = plsc.get_sparse_core_info()
    nw = info.num_cores * info.num_subcores
    per_w = B // nw
    mesh = plsc.VectorSubcoreMesh(core_axis_name="c", subcore_axis_name="s")

    @functools.partial(
        pl.kernel, mesh=mesh,
        out_type=jax.ShapeDtypeStruct((B, D), jnp.float32),
        scratch_types=[
            pltpu.VMEM((_CHUNK,), jnp.int32),
            pltpu.VMEM((_CHUNK, D), jnp.float32),
            pltpu.SemaphoreType.DMA,
        ],
    )
    def k(flat_hbm, slot_hbm, out_hbm, idx_v, rows_v, sem):
        wid = lax.axis_index("s") * info.num_cores + lax.axis_index("c")
        for c in range(per_w // _CHUNK):
            base = wid * per_w + c * _CHUNK
            pltpu.sync_copy(slot_hbm.at[pl.ds(base, _CHUNK)], idx_v)
            pltpu.async_copy(flat_hbm.at[idx_v], rows_v, sem).wait()
            pltpu.sync_copy(rows_v, out_hbm.at[pl.ds(base, _CHUNK)])

    return k(flat, slot_g)


def kernel(x, conv1_w, conv1_b, conv2_w, conv2_b, fc1_w, fc1_b,
           fc2_w, fc2_b, gate_w, expert_w, expert_b, fc4_w, fc4_b):
    x2 = x.transpose(0, 2, 1, 3).reshape(B, 32, 96)
    xp = [x2[:, q::4] for q in range(4)]
    w1p = _build_w1p(conv1_w)
    b1p = jnp.broadcast_to(conv1_b[None, :, None], (2, 6, 14)).reshape(1, 168)
    w2p = _build_w2p(conv2_w)
    b2p = jnp.broadcast_to(conv2_b[None, :, None], (2, 16, 5)).reshape(1, 160)
    fc1_wt = (fc1_w.reshape(4096, 16, 5, 5).transpose(0, 2, 1, 3)
              .reshape(4096, 400).T)
    fc2_wt = fc2_w.T
    fc4_wt = fc4_w.T
    expert_b3 = expert_b.reshape(E, 1, D)

    feat = _conv_trunk(xp, w1p, b1p, w2p, b2p)
    h1 = _fc1(feat, fc1_wt, fc1_b.reshape(1, 4096))
    h = _fc2(h1, fc2_wt, fc2_b.reshape(1, D))
    slot_s, slot_g, coef, keep = _route(h, gate_w)
    buf = _sc_dispatch(h, slot_s.reshape(B))
    flat = _experts(buf, expert_w, expert_b3)
    gathered = _sc_combine(flat, slot_g.reshape(B))
    return _fc4(gathered, coef, keep, fc4_wt, fc4_b.reshape(1, 10))
